# baseline (speedup 1.0000x reference)
import functools
import math

import jax
import jax.numpy as jnp
from jax import lax
from jax.experimental import pallas as pl
from jax.experimental.pallas import tpu as pltpu
from jax.experimental.pallas import tpu_sc as plsc

NUM_TOKENS = 1000000
DIM = 64
SCALE = float(math.sqrt(DIM))

_info = plsc.get_sparse_core_info()
NC, NS, L = _info.num_cores, _info.num_subcores, _info.num_lanes
NW = NC * NS

GROUP = 128
KC = 4
CHUNK = KC * GROUP


@functools.partial(jax.jit, static_argnames=("total_rows",))
def _emb_lookup(xg, table, *, total_rows):
    groups_per_w = (total_rows // GROUP) // NW
    chunks_per_w = groups_per_w // KC

    mesh = plsc.VectorSubcoreMesh(core_axis_name="c", subcore_axis_name="s")

    @functools.partial(
        pl.kernel,
        mesh=mesh,
        out_type=jax.ShapeDtypeStruct((total_rows, DIM), jnp.float32),
        scratch_types=[
            pltpu.VMEM((KC, GROUP), jnp.int32),
            pltpu.VMEM((CHUNK, DIM), jnp.float32),
            pltpu.SemaphoreType.DMA,
        ],
        compiler_params=pltpu.CompilerParams(use_tc_tiling_on_sc=False),
    )
    def k(x_hbm, tab_hbm, out_hbm, idx_v, rows_v, sem):
        wid = lax.axis_index("s") * NC + lax.axis_index("c")
        g0 = wid * groups_per_w

        def chunk_body(g, carry):
            base_g = g0 + g * KC
            pltpu.sync_copy(x_hbm.at[pl.ds(base_g, KC)], idx_v)
            copies = []
            for j in range(KC):
                copies.append(
                    pltpu.async_copy(
                        tab_hbm.at[idx_v.at[j]],
                        rows_v.at[pl.ds(j * GROUP, GROUP)],
                        sem,
                    )
                )
            for c in copies:
                c.wait()

            def scale_body(i, carry2):
                for c in range(DIM // L):
                    sl = pl.ds(c * L, L)
                    rows_v[i, sl] = rows_v[i, sl] * SCALE
                return carry2

            lax.fori_loop(0, CHUNK, scale_body, 0)
            pltpu.sync_copy(rows_v, out_hbm.at[pl.ds(base_g * GROUP, CHUNK)])
            return carry

        lax.fori_loop(0, chunks_per_w, chunk_body, 0)

    return k(xg, table)


def kernel(x, table):
    b, h = x.shape
    total = b * h
    xg = x.reshape(total // GROUP, GROUP)
    out = _emb_lookup(xg, table, total_rows=total)
    return out.reshape(b, h, DIM)


# speedup vs baseline: 1.1164x; 1.1164x over previous
import functools
import math

import jax
import jax.numpy as jnp
from jax import lax
from jax.experimental import pallas as pl
from jax.experimental.pallas import tpu as pltpu
from jax.experimental.pallas import tpu_sc as plsc

NUM_TOKENS = 1000000
DIM = 64
SCALE = float(math.sqrt(DIM))

_info = plsc.get_sparse_core_info()
NC, NS, L = _info.num_cores, _info.num_subcores, _info.num_lanes
NW = NC * NS

GROUP = 128
KC = 4
CHUNK = KC * GROUP


@functools.partial(jax.jit, static_argnames=("total_rows",))
def _emb_lookup(xg, table, *, total_rows):
    groups_per_w = (total_rows // GROUP) // NW
    chunks_per_w = groups_per_w // KC
    assert chunks_per_w % 2 == 0 and chunks_per_w >= 4

    mesh = plsc.VectorSubcoreMesh(core_axis_name="c", subcore_axis_name="s")

    @functools.partial(
        pl.kernel,
        mesh=mesh,
        out_type=jax.ShapeDtypeStruct((total_rows, DIM), jnp.float32),
        scratch_types=[
            pltpu.VMEM((KC, GROUP), jnp.int32),
            pltpu.VMEM((KC, GROUP), jnp.int32),
            pltpu.VMEM((CHUNK, DIM), jnp.float32),
            pltpu.VMEM((CHUNK, DIM), jnp.float32),
            pltpu.SemaphoreType.DMA,
            pltpu.SemaphoreType.DMA,
            pltpu.SemaphoreType.DMA,
            pltpu.SemaphoreType.DMA,
        ],
        compiler_params=pltpu.CompilerParams(use_tc_tiling_on_sc=False),
    )
    def k(x_hbm, tab_hbm, out_hbm, idx0, idx1, rows0, rows1, sg0, sg1, sw0, sw1):
        wid = lax.axis_index("s") * NC + lax.axis_index("c")
        g0 = wid * groups_per_w
        idx = (idx0, idx1)
        rows = (rows0, rows1)
        sg = (sg0, sg1)
        sw = (sw0, sw1)

        def fire_gather(g, b):
            base_g = g0 + g * KC
            pltpu.sync_copy(x_hbm.at[pl.ds(base_g, KC)], idx[b])
            for j in range(KC):
                pltpu.async_copy(
                    tab_hbm.at[idx[b].at[j]],
                    rows[b].at[pl.ds(j * GROUP, GROUP)],
                    sg[b],
                )

        def wait_gather(b):
            pltpu.make_async_copy(out_hbm.at[pl.ds(0, CHUNK)], rows[b], sg[b]).wait()

        def wait_write(b):
            pltpu.make_async_copy(rows[b], out_hbm.at[pl.ds(0, CHUNK)], sw[b]).wait()

        def scale(b):
            @pl.loop(0, CHUNK, unroll=8)
            def _(i):
                for c in range(DIM // L):
                    sl = pl.ds(c * L, L)
                    rows[b][i, sl] = rows[b][i, sl] * SCALE

        def fire_write(g, b):
            base = (g0 + g * KC) * GROUP
            pltpu.async_copy(rows[b], out_hbm.at[pl.ds(base, CHUNK)], sw[b])

        fire_gather(0, 0)

        @pl.loop(0, chunks_per_w, step=2)
        def _(t):
            for b in range(2):
                g = t + b
                nb = 1 - b
                g1 = g + 1

                @pl.when(g1 < chunks_per_w)
                def _():
                    @pl.when(g1 >= 2)
                    def _():
                        wait_write(nb)

                    fire_gather(g1, nb)

                wait_gather(b)
                scale(b)
                fire_write(g, b)

        wait_write(0)
        wait_write(1)

    return k(xg, table)


def kernel(x, table):
    b, h = x.shape
    total = b * h
    xg = x.reshape(total // GROUP, GROUP)
    out = _emb_lookup(xg, table, total_rows=total)
    return out.reshape(b, h, DIM)


# speedup vs baseline: 1.1328x; 1.0147x over previous
import functools
import math

import jax
import jax.numpy as jnp
from jax import lax
from jax.experimental import pallas as pl
from jax.experimental.pallas import tpu as pltpu
from jax.experimental.pallas import tpu_sc as plsc

NUM_TOKENS = 1000000
DIM = 64
SCALE = float(math.sqrt(DIM))

_info = plsc.get_sparse_core_info()
NC, NS, L = _info.num_cores, _info.num_subcores, _info.num_lanes
NW = NC * NS

LB = 128
U = 2
RT = DIM // 8


@functools.partial(jax.jit, static_argnames=("batch", "hist"))
def _emb_lookup(xt, table, *, batch, hist):
    cblocks = batch // LB
    su_total = hist * (cblocks // U)
    su_per_w = su_total // NW
    assert su_total % NW == 0 and su_per_w % 2 == 0
    cb_per_h = cblocks // U

    mesh = plsc.VectorSubcoreMesh(core_axis_name="c", subcore_axis_name="s")

    @functools.partial(
        pl.kernel,
        mesh=mesh,
        out_type=jax.ShapeDtypeStruct((hist, RT, cblocks, 8, LB), jnp.float32),
        scratch_types=[
            pltpu.VMEM((U, LB), jnp.int32),
            pltpu.VMEM((U, LB), jnp.int32),
            pltpu.VMEM((U * LB, DIM), jnp.float32),
            pltpu.VMEM((U * LB, DIM), jnp.float32),
            pltpu.VMEM((U * DIM, LB), jnp.float32),
            pltpu.VMEM((U * DIM, LB), jnp.float32),
            pltpu.SemaphoreType.DMA,
            pltpu.SemaphoreType.DMA,
            pltpu.SemaphoreType.DMA,
            pltpu.SemaphoreType.DMA,
        ],
        compiler_params=pltpu.CompilerParams(
            use_tc_tiling_on_sc=False, needs_layout_passes=False
        ),
    )
    def k(xt_hbm, tab_hbm, out_hbm, i0, i1, g0, g1, t0, t1, sg0, sg1, sw0, sw1):
        wid = lax.axis_index("s") * NC + lax.axis_index("c")
        su0 = wid * su_per_w
        idx = (i0, i1)
        G = (g0, g1)
        TR = (t0, t1)
        sg = (sg0, sg1)
        sw = (sw0, sw1)

        iota = lax.iota(jnp.int32, L)
        dvecs = [lax.rem(iota + k, jnp.int32(16)) for k in range(16)]

        def fire(su, b):
            h = su // cb_per_h
            c0 = (su % cb_per_h) * U
            pltpu.sync_copy(xt_hbm.at[h, pl.ds(c0, U)], idx[b])
            for u in range(U):
                pltpu.async_copy(
                    tab_hbm.at[idx[b].at[u]],
                    G[b].at[pl.ds(u * LB, LB)],
                    sg[b],
                )

        def wait_gathers(b):
            for _ in range(U):
                pltpu.make_async_copy(
                    tab_hbm.at[pl.ds(0, LB)], G[b].at[pl.ds(0, LB)], sg[b]
                ).wait()

        def wait_writes(b):
            for _ in range(U * RT):
                pltpu.make_async_copy(
                    TR[b].at[pl.ds(0, 8)], out_hbm.at[0, 0, 0], sw[b]
                ).wait()

        def transpose_scale(b):
            @pl.loop(0, U * (LB // 16) * (DIM // 16))
            def _(blk):
                u = blk // ((LB // 16) * (DIM // 16))
                rest = blk % ((LB // 16) * (DIM // 16))
                l0 = (rest // (DIM // 16)) * 16
                d0 = (rest % (DIM // 16)) * 16
                lvec_g = iota + (u * LB + l0)
                lvec_t = iota + l0
                for kk in range(16):
                    dvec_g = dvecs[kk] + d0
                    dvec_t = dvecs[kk] + (u * DIM + d0)
                    vals = plsc.load_gather(G[b], [lvec_g, dvec_g])
                    plsc.store_scatter(TR[b], [dvec_t, lvec_t], vals * SCALE)

        def fire_writes(su, b):
            h = su // cb_per_h
            c0 = (su % cb_per_h) * U
            for u in range(U):
                for r in range(RT):
                    pltpu.async_copy(
                        TR[b].at[pl.ds(u * DIM + r * 8, 8)],
                        out_hbm.at[h, r, c0 + u],
                        sw[b],
                    )

        fire(su0, 0)

        @pl.loop(0, su_per_w, step=2)
        def _(t):
            for b in range(2):
                su = su0 + t + b
                nb = 1 - b

                @pl.when(t + b + 1 < su_per_w)
                def _():
                    @pl.when(t + b >= 1)
                    def _():
                        wait_writes(nb)

                    fire(su + 1, nb)

                wait_gathers(b)
                transpose_scale(b)
                fire_writes(su, b)

        wait_writes(0)
        wait_writes(1)

    return k(xt, table)


def kernel(x, table):
    b, h = x.shape
    xt = jnp.transpose(x).reshape(h, b // LB, LB)
    ot = _emb_lookup(xt, table, batch=b, hist=h)
    out = jnp.transpose(ot, (2, 4, 0, 1, 3)).reshape(b, h, DIM)
    return out


# speedup vs baseline: 1.1782x; 1.0400x over previous
import functools
import math

import jax
import jax.numpy as jnp
from jax import lax
from jax.experimental import pallas as pl
from jax.experimental.pallas import tpu as pltpu
from jax.experimental.pallas import tpu_sc as plsc

NUM_TOKENS = 1000000
DIM = 64
SCALE = float(math.sqrt(DIM))

_info = plsc.get_sparse_core_info()
NC, NS, L = _info.num_cores, _info.num_subcores, _info.num_lanes
NW = NC * NS

LB = 128
LBP = LB + 1
U = 2
RT = DIM // 8


@functools.partial(jax.jit, static_argnames=("batch", "hist"))
def _emb_lookup(xt, table, *, batch, hist):
    cblocks = batch // LB
    su_total = hist * (cblocks // U)
    su_per_w = su_total // NW
    assert su_total % NW == 0 and su_per_w % 2 == 0
    cb_per_h = cblocks // U

    mesh = plsc.VectorSubcoreMesh(core_axis_name="c", subcore_axis_name="s")

    @functools.partial(
        pl.kernel,
        mesh=mesh,
        out_type=jax.ShapeDtypeStruct((hist, RT, cblocks, 8, LB), jnp.float32),
        scratch_types=[
            pltpu.VMEM((U, LB), jnp.int32),
            pltpu.VMEM((U, LB), jnp.int32),
            pltpu.VMEM((U * LB, DIM), jnp.float32),
            pltpu.VMEM((U * LB, DIM), jnp.float32),
            pltpu.VMEM((U * DIM, LBP), jnp.float32),
            pltpu.VMEM((U * DIM, LBP), jnp.float32),
            pltpu.SemaphoreType.DMA,
            pltpu.SemaphoreType.DMA,
            pltpu.SemaphoreType.DMA,
            pltpu.SemaphoreType.DMA,
        ],
        compiler_params=pltpu.CompilerParams(
            use_tc_tiling_on_sc=False, needs_layout_passes=False
        ),
    )
    def k(xt_hbm, tab_hbm, out_hbm, i0, i1, g0, g1, t0, t1, sg0, sg1, sw0, sw1):
        wid = lax.axis_index("s") * NC + lax.axis_index("c")
        su0 = wid * su_per_w
        idx = (i0, i1)
        G = (g0, g1)
        TR = (t0, t1)
        sg = (sg0, sg1)
        sw = (sw0, sw1)

        iota = lax.iota(jnp.int32, L)
        rvecs = [
            [iota + (u * DIM + dc * L) for dc in range(DIM // L)]
            for u in range(U)
        ]

        def fire(su, b):
            h = su // cb_per_h
            c0 = (su % cb_per_h) * U
            pltpu.sync_copy(xt_hbm.at[h, pl.ds(c0, U)], idx[b])
            for u in range(U):
                pltpu.async_copy(
                    tab_hbm.at[idx[b].at[u]],
                    G[b].at[pl.ds(u * LB, LB)],
                    sg[b],
                )

        def wait_gathers(b):
            for _ in range(U):
                pltpu.make_async_copy(
                    tab_hbm.at[pl.ds(0, LB)], G[b].at[pl.ds(0, LB)], sg[b]
                ).wait()

        def wait_writes(b):
            for _ in range(U * RT):
                pltpu.make_async_copy(
                    TR[b].at[pl.ds(0, 8), pl.ds(0, LB)],
                    out_hbm.at[0, 0, 0],
                    sw[b],
                ).wait()

        def transpose_scale(b):
            for u in range(U):

                @pl.loop(0, LB, unroll=4)
                def _(l):
                    row = u * LB + l
                    col = jnp.broadcast_to(l, (L,)).astype(jnp.int32)
                    for dc in range(DIM // L):
                        vals = G[b][row, pl.ds(dc * L, L)]
                        plsc.store_scatter(
                            TR[b], [rvecs[u][dc], col], vals * SCALE
                        )

        def fire_writes(su, b):
            h = su // cb_per_h
            c0 = (su % cb_per_h) * U
            for u in range(U):
                for r in range(RT):
                    pltpu.async_copy(
                        TR[b].at[pl.ds(u * DIM + r * 8, 8), pl.ds(0, LB)],
                        out_hbm.at[h, r, c0 + u],
                        sw[b],
                    )

        fire(su0, 0)

        @pl.loop(0, su_per_w, step=2)
        def _(t):
            for b in range(2):
                su = su0 + t + b
                nb = 1 - b

                @pl.when(t + b + 1 < su_per_w)
                def _():
                    @pl.when(t + b >= 1)
                    def _():
                        wait_writes(nb)

                    fire(su + 1, nb)

                wait_gathers(b)
                transpose_scale(b)
                fire_writes(su, b)

        wait_writes(0)
        wait_writes(1)

    return k(xt, table)


def kernel(x, table):
    b, h = x.shape
    xt = jnp.transpose(x).reshape(h, b // LB, LB)
    ot = _emb_lookup(xt, table, batch=b, hist=h)
    out = jnp.transpose(ot, (2, 4, 0, 1, 3)).reshape(b, h, DIM)
    return out


# speedup vs baseline: 1.7642x; 1.4974x over previous
import functools
import math

import jax
import jax.numpy as jnp
from jax import lax
from jax.experimental import pallas as pl
from jax.experimental.pallas import tpu as pltpu
from jax.experimental.pallas import tpu_sc as plsc

NUM_TOKENS = 1000000
DIM = 64
SCALE = float(math.sqrt(DIM))

_info = plsc.get_sparse_core_info()
NC, NS, L = _info.num_cores, _info.num_subcores, _info.num_lanes
NW = NC * NS

LB = 128
LBP = LB + 1
U = 2
RT = DIM // 8


@functools.partial(jax.jit, static_argnames=("batch", "hist"))
def _emb_lookup(xt, table, *, batch, hist):
    cblocks = batch // LB
    su_total = hist * (cblocks // U)
    su_per_w = su_total // NW
    assert su_total % NW == 0 and su_per_w % 2 == 0
    cb_per_h = cblocks // U

    mesh = plsc.VectorSubcoreMesh(core_axis_name="c", subcore_axis_name="s")

    @functools.partial(
        pl.kernel,
        mesh=mesh,
        out_type=jax.ShapeDtypeStruct((hist, RT, cblocks, 8, LB), jnp.float32),
        scratch_types=[
            pltpu.VMEM((U, LB), jnp.int32),
            pltpu.VMEM((U, LB), jnp.int32),
            pltpu.VMEM((U * LB, DIM), jnp.float32),
            pltpu.VMEM((U * LB, DIM), jnp.float32),
            pltpu.VMEM((U * DIM, LBP), jnp.float32),
            pltpu.VMEM((U * DIM, LBP), jnp.float32),
            pltpu.SemaphoreType.DMA,
            pltpu.SemaphoreType.DMA,
            pltpu.SemaphoreType.DMA,
            pltpu.SemaphoreType.DMA,
        ],
        compiler_params=pltpu.CompilerParams(
            use_tc_tiling_on_sc=False, needs_layout_passes=False
        ),
    )
    def k(xt_hbm, tab_hbm, out_hbm, i0, i1, g0, g1, t0, t1, sg0, sg1, sw0, sw1):
        wid = lax.axis_index("s") * NC + lax.axis_index("c")
        su0 = wid * su_per_w
        idx = (i0, i1)
        G = (g0, g1)
        TR = (t0, t1)
        sg = (sg0, sg1)
        sw = (sw0, sw1)

        iota = lax.iota(jnp.int32, L)
        rvecs = [
            [iota + (u * DIM + dc * L) for dc in range(DIM // L)]
            for u in range(U)
        ]

        def fire(su, b):
            h = su // cb_per_h
            c0 = (su % cb_per_h) * U
            pltpu.sync_copy(xt_hbm.at[h, pl.ds(c0, U)], idx[b])
            for u in range(U):
                pltpu.async_copy(
                    tab_hbm.at[idx[b].at[u]],
                    G[b].at[pl.ds(u * LB, LB)],
                    sg[b],
                )

        def wait_gathers(b):
            for _ in range(U):
                pltpu.make_async_copy(
                    tab_hbm.at[pl.ds(0, LB)], G[b].at[pl.ds(0, LB)], sg[b]
                ).wait()

        def wait_writes(b):
            for _ in range(U * RT):
                pltpu.make_async_copy(
                    TR[b].at[pl.ds(0, 8), pl.ds(0, LB)],
                    out_hbm.at[0, 0, 0],
                    sw[b],
                ).wait()

        def transpose_scale(b):
            for u in range(U):

                @plsc.parallel_loop(0, LB, unroll=4)
                def _(l):
                    row = u * LB + l
                    col = jnp.broadcast_to(l, (L,)).astype(jnp.int32)
                    for dc in range(DIM // L):
                        vals = G[b][row, pl.ds(dc * L, L)]
                        plsc.store_scatter(
                            TR[b], [rvecs[u][dc], col], vals * SCALE
                        )

        def fire_writes(su, b):
            h = su // cb_per_h
            c0 = (su % cb_per_h) * U
            for u in range(U):
                for r in range(RT):
                    pltpu.async_copy(
                        TR[b].at[pl.ds(u * DIM + r * 8, 8), pl.ds(0, LB)],
                        out_hbm.at[h, r, c0 + u],
                        sw[b],
                    )

        fire(su0, 0)

        @pl.loop(0, su_per_w, step=2)
        def _(t):
            for b in range(2):
                su = su0 + t + b
                nb = 1 - b

                @pl.when(t + b + 1 < su_per_w)
                def _():
                    @pl.when(t + b >= 1)
                    def _():
                        wait_writes(nb)

                    fire(su + 1, nb)

                wait_gathers(b)
                transpose_scale(b)
                fire_writes(su, b)

        wait_writes(0)
        wait_writes(1)

    return k(xt, table)


def kernel(x, table):
    b, h = x.shape
    xt = jnp.transpose(x).reshape(h, b // LB, LB)
    ot = _emb_lookup(xt, table, batch=b, hist=h)
    out = jnp.transpose(ot, (2, 4, 0, 1, 3)).reshape(b, h, DIM)
    return out
